# Initial kernel scaffold; baseline (speedup 1.0000x reference)
import jax
import jax.numpy as jnp
from jax.experimental import pallas as pl


def kernel(x, edge_index, output_node_mask, W0, ln0_g, ln0_b, W1, ln1_g, ln1_b, Wl, bl):
    raise NotImplementedError("write your pallas kernel here")


# speedup vs baseline: 2.5852x; 2.5852x over previous
import functools

import jax
import jax.numpy as jnp
from jax import lax
from jax.experimental import pallas as pl
from jax.experimental.pallas import tpu as pltpu
from jax.experimental.pallas import tpu_sc as plsc

N = 10000
E = 320000
D = 128
H = 128
C = 40
P = 1024

NP = 10240
NC = 2
NS = 16
NW = NC * NS
CHUNK = 128
EPT_CHUNKS = 80
EPT = EPT_CHUNKS * CHUNK
E_PAD = EPT * NW
ROWS_PER_SUB = NP // NS
P_PER_SUB = P // NS
EPS = 1e-5


def _sc_mesh():
  return plsc.VectorSubcoreMesh(core_axis_name="c", subcore_axis_name="s")


def _agg_body(prime, h_ref, src_ref, dst_ref, z_ref, mask_ref, out_ref,
              sidx, didx, rows, midx, mrows, acc, sem):
  c = lax.axis_index("c")
  s = lax.axis_index("s")
  wid = c * NS + s

  pltpu.sync_copy(z_ref, acc.at[pl.ds(s * ROWS_PER_SUB, ROWS_PER_SUB)])

  pltpu.sync_copy(src_ref.at[pl.ds(wid * EPT_CHUNKS, EPT_CHUNKS)], sidx)
  pltpu.sync_copy(dst_ref.at[pl.ds(wid * EPT_CHUNKS, EPT_CHUNKS)], didx)
  plsc.subcore_barrier()

  def chunk(j, carry):
    pltpu.async_copy(h_ref.at[sidx.at[j]], rows, sem).wait()
    pltpu.sync_copy(rows, acc.at[didx.at[j]], add=True)
    return carry

  lax.fori_loop(0, EPT_CHUNKS, chunk, 0)
  plsc.subcore_barrier()

  if prime:
    pltpu.sync_copy(mask_ref.at[pl.ds(s * P_PER_SUB, P_PER_SUB)], midx)
    pltpu.async_copy(acc.at[midx], mrows, sem).wait()
    pltpu.sync_copy(mrows, out_ref.at[pl.ds(c * P + s * P_PER_SUB, P_PER_SUB)])
  else:
    pltpu.sync_copy(acc.at[pl.ds(s * ROWS_PER_SUB, ROWS_PER_SUB)],
                    out_ref.at[pl.ds(c * NP + s * ROWS_PER_SUB, ROWS_PER_SUB)])


def _make_agg(prime):
  out_rows = 2 * P if prime else 2 * NP
  body = functools.partial(_agg_body, prime)
  return pl.kernel(
      body,
      out_type=jax.ShapeDtypeStruct((out_rows, H), jnp.float32),
      mesh=_sc_mesh(),
      scratch_types=[
          pltpu.VMEM((EPT_CHUNKS, CHUNK), jnp.int32),
          pltpu.VMEM((EPT_CHUNKS, CHUNK), jnp.int32),
          pltpu.VMEM((CHUNK, H), jnp.float32),
          pltpu.VMEM((P_PER_SUB,), jnp.int32),
          pltpu.VMEM((P_PER_SUB, H), jnp.float32),
          pltpu.VMEM_SHARED((NP, H), jnp.float32),
          pltpu.SemaphoreType.DMA,
      ],
      name="sc_edge_agg_prime" if prime else "sc_edge_agg",
  )


def _mm_body(x_ref, w_ref, o_ref):
  o_ref[...] = jnp.dot(x_ref[...], w_ref[...],
                       preferred_element_type=jnp.float32)


def _tc_matmul(x, w):
  m = x.shape[0]
  bm = 1280
  grid = m // bm
  return pl.pallas_call(
      _mm_body,
      grid=(grid,),
      in_specs=[
          pl.BlockSpec((bm, D), lambda i: (i, 0)),
          pl.BlockSpec((D, H), lambda i: (0, 0)),
      ],
      out_specs=pl.BlockSpec((bm, H), lambda i: (i, 0)),
      out_shape=jax.ShapeDtypeStruct((m, H), jnp.float32),
  )(x, w)


def _ln_relu(a, g, b):
  mu = jnp.mean(a, axis=-1, keepdims=True)
  var = jnp.mean((a - mu) ** 2, axis=-1, keepdims=True)
  hn = (a - mu) * lax.rsqrt(var + EPS) * g + b
  return jnp.maximum(hn, 0.0)


def _mid_body(p0_ref, p1_ref, g_ref, b_ref, w_ref, o_ref):
  a = p0_ref[...] + p1_ref[...]
  h = _ln_relu(a, g_ref[...], b_ref[...])
  o_ref[...] = jnp.dot(h, w_ref[...], preferred_element_type=jnp.float32)


def _tc_mid(p0, p1, g, b, w):
  m = p0.shape[0]
  bm = 1280
  grid = m // bm
  return pl.pallas_call(
      _mid_body,
      grid=(grid,),
      in_specs=[
          pl.BlockSpec((bm, H), lambda i: (i, 0)),
          pl.BlockSpec((bm, H), lambda i: (i, 0)),
          pl.BlockSpec((1, H), lambda i: (0, 0)),
          pl.BlockSpec((1, H), lambda i: (0, 0)),
          pl.BlockSpec((H, H), lambda i: (0, 0)),
      ],
      out_specs=pl.BlockSpec((bm, H), lambda i: (i, 0)),
      out_shape=jax.ShapeDtypeStruct((m, H), jnp.float32),
  )(p0, p1, g, b, w)


def _final_body(p0_ref, p1_ref, g_ref, b_ref, w_ref, bl_ref, o_ref):
  a = p0_ref[...] + p1_ref[...]
  h = _ln_relu(a, g_ref[...], b_ref[...])
  z = jnp.dot(h, w_ref[...], preferred_element_type=jnp.float32) + bl_ref[...]
  col = lax.broadcasted_iota(jnp.int32, z.shape, 1)
  z = jnp.where(col < C, z, -1e30)
  m = jnp.max(z, axis=-1, keepdims=True)
  lse = jnp.log(jnp.sum(jnp.exp(z - m), axis=-1, keepdims=True))
  o_ref[...] = z - m - lse


def _tc_final(p0, p1, g, b, w, bl):
  return pl.pallas_call(
      _final_body,
      out_shape=jax.ShapeDtypeStruct((P, H), jnp.float32),
  )(p0, p1, g, b, w, bl)


def kernel(x, edge_index, output_node_mask, W0, ln0_g, ln0_b, W1, ln1_g,
           ln1_b, Wl, bl):
  xp = jnp.pad(x, ((0, NP - N), (0, 0)))
  dst = jnp.concatenate(
      [edge_index[0], jnp.full((E_PAD - E,), NP - 1, jnp.int32)])
  src = jnp.concatenate([edge_index[1], jnp.zeros((E_PAD - E,), jnp.int32)])
  src2d = src.reshape(E_PAD // CHUNK, CHUNK)
  dst2d = dst.reshape(E_PAD // CHUNK, CHUNK)
  zeros = jnp.zeros((ROWS_PER_SUB, H), jnp.float32)
  dummy_mask = jnp.zeros((P,), jnp.int32)

  g0 = ln0_g.reshape(1, H)
  b0 = ln0_b.reshape(1, H)
  g1 = ln1_g.reshape(1, H)
  b1 = ln1_b.reshape(1, H)
  Wl_pad = jnp.pad(Wl, ((0, 0), (0, H - C)))
  bl_pad = jnp.pad(bl, ((0, H - C))).reshape(1, H)

  h = _tc_matmul(xp, W0)
  parts = _make_agg(False)(h, src2d, dst2d, zeros, dummy_mask)
  h2 = _tc_mid(parts[:NP], parts[NP:], g0, b0, W1)
  parts2 = _make_agg(True)(h2, src2d, dst2d, zeros, output_node_mask)
  outp = _tc_final(parts2[:P], parts2[P:], g1, b1, Wl_pad, bl_pad)
  return outp[:, :C]


# speedup vs baseline: 3.0096x; 1.1641x over previous
import functools

import jax
import jax.numpy as jnp
from jax import lax
from jax.experimental import pallas as pl
from jax.experimental.pallas import tpu as pltpu
from jax.experimental.pallas import tpu_sc as plsc

N = 10000
E = 320000
D = 128
H = 128
C = 40
P = 1024

NP = 10240
NC = 2
NS = 16
NW = NC * NS
CHUNK = 128
EPT_CHUNKS = 80
EPT = EPT_CHUNKS * CHUNK
E_PAD = EPT * NW
ROWS_PER_SUB = NP // NS
P_PER_SUB = P // NS
EPS = 1e-5


def _sc_mesh():
  return plsc.VectorSubcoreMesh(core_axis_name="c", subcore_axis_name="s")


def _unpack(ed, j, sidx, didx):
  row = ed.at[j]
  for k in range(CHUNK // 16):
    v = row[pl.ds(16 * k, 16)]
    sidx[pl.ds(16 * k, 16)] = lax.bitwise_and(v, 0xFFFF)
    didx[pl.ds(16 * k, 16)] = lax.shift_right_logical(v, 16)


def _agg_body(prime, h_ref, ed_ref, z_ref, mask_ref, out_ref,
              ed, sidx_a, didx_a, sidx_b, didx_b, rows_a, rows_b,
              acc, sem_a, sem_b):
  c = lax.axis_index("c")
  s = lax.axis_index("s")
  wid = c * NS + s

  pltpu.sync_copy(z_ref, acc.at[pl.ds(s * ROWS_PER_SUB, ROWS_PER_SUB)])

  pltpu.sync_copy(ed_ref.at[pl.ds(wid * EPT_CHUNKS, EPT_CHUNKS)], ed)
  plsc.subcore_barrier()

  last = EPT_CHUNKS - 1
  _unpack(ed, 0, sidx_a, didx_a)
  pltpu.async_copy(h_ref.at[sidx_a], rows_a, sem_a)

  def pair(j, carry):
    c0 = 2 * j
    c1 = c0 + 1
    c2 = jnp.minimum(c0 + 2, last)
    _unpack(ed, c1, sidx_b, didx_b)
    pltpu.async_copy(h_ref.at[sidx_b], rows_b, sem_b)
    pltpu.make_async_copy(h_ref.at[sidx_a], rows_a, sem_a).wait()
    pltpu.sync_copy(rows_a, acc.at[didx_a], add=True)
    _unpack(ed, c2, sidx_a, didx_a)
    pltpu.async_copy(h_ref.at[sidx_a], rows_a, sem_a)
    pltpu.make_async_copy(h_ref.at[sidx_b], rows_b, sem_b).wait()
    pltpu.sync_copy(rows_b, acc.at[didx_b], add=True)
    return carry

  lax.fori_loop(0, EPT_CHUNKS // 2, pair, 0)
  pltpu.make_async_copy(h_ref.at[sidx_a], rows_a, sem_a).wait()
  plsc.subcore_barrier()

  if prime:
    midx = sidx_a.at[pl.ds(0, P_PER_SUB)]
    mrows = rows_a.at[pl.ds(0, P_PER_SUB)]
    pltpu.sync_copy(mask_ref.at[pl.ds(s * P_PER_SUB, P_PER_SUB)], midx)
    pltpu.async_copy(acc.at[midx], mrows, sem_a).wait()
    pltpu.sync_copy(mrows, out_ref.at[pl.ds(c * P + s * P_PER_SUB, P_PER_SUB)])
  else:
    pltpu.sync_copy(acc.at[pl.ds(s * ROWS_PER_SUB, ROWS_PER_SUB)],
                    out_ref.at[pl.ds(c * NP + s * ROWS_PER_SUB, ROWS_PER_SUB)])


def _make_agg(prime):
  out_rows = 2 * P if prime else 2 * NP
  body = functools.partial(_agg_body, prime)
  return pl.kernel(
      body,
      out_type=jax.ShapeDtypeStruct((out_rows, H), jnp.float32),
      mesh=_sc_mesh(),
      scratch_types=[
          pltpu.VMEM((EPT_CHUNKS, CHUNK), jnp.int32),
          pltpu.VMEM((CHUNK,), jnp.int32),
          pltpu.VMEM((CHUNK,), jnp.int32),
          pltpu.VMEM((CHUNK,), jnp.int32),
          pltpu.VMEM((CHUNK,), jnp.int32),
          pltpu.VMEM((CHUNK, H), jnp.float32),
          pltpu.VMEM((CHUNK, H), jnp.float32),
          pltpu.VMEM_SHARED((NP, H), jnp.float32),
          pltpu.SemaphoreType.DMA,
          pltpu.SemaphoreType.DMA,
      ],
      name="sc_edge_agg_prime" if prime else "sc_edge_agg",
  )


def _mm_body(x_ref, w_ref, o_ref):
  o_ref[...] = jnp.dot(x_ref[...], w_ref[...],
                       preferred_element_type=jnp.float32)


def _tc_matmul(x, w):
  m = x.shape[0]
  bm = 1280
  grid = m // bm
  return pl.pallas_call(
      _mm_body,
      grid=(grid,),
      in_specs=[
          pl.BlockSpec((bm, D), lambda i: (i, 0)),
          pl.BlockSpec((D, H), lambda i: (0, 0)),
      ],
      out_specs=pl.BlockSpec((bm, H), lambda i: (i, 0)),
      out_shape=jax.ShapeDtypeStruct((m, H), jnp.float32),
  )(x, w)


def _ln_relu(a, g, b):
  mu = jnp.mean(a, axis=-1, keepdims=True)
  var = jnp.mean((a - mu) ** 2, axis=-1, keepdims=True)
  hn = (a - mu) * lax.rsqrt(var + EPS) * g + b
  return jnp.maximum(hn, 0.0)


def _mid_body(p0_ref, p1_ref, g_ref, b_ref, w_ref, o_ref):
  a = p0_ref[...] + p1_ref[...]
  h = _ln_relu(a, g_ref[...], b_ref[...])
  o_ref[...] = jnp.dot(h, w_ref[...], preferred_element_type=jnp.float32)


def _tc_mid(p0, p1, g, b, w):
  m = p0.shape[0]
  bm = 1280
  grid = m // bm
  return pl.pallas_call(
      _mid_body,
      grid=(grid,),
      in_specs=[
          pl.BlockSpec((bm, H), lambda i: (i, 0)),
          pl.BlockSpec((bm, H), lambda i: (i, 0)),
          pl.BlockSpec((1, H), lambda i: (0, 0)),
          pl.BlockSpec((1, H), lambda i: (0, 0)),
          pl.BlockSpec((H, H), lambda i: (0, 0)),
      ],
      out_specs=pl.BlockSpec((bm, H), lambda i: (i, 0)),
      out_shape=jax.ShapeDtypeStruct((m, H), jnp.float32),
  )(p0, p1, g, b, w)


def _final_body(p0_ref, p1_ref, g_ref, b_ref, w_ref, bl_ref, o_ref):
  a = p0_ref[...] + p1_ref[...]
  h = _ln_relu(a, g_ref[...], b_ref[...])
  z = jnp.dot(h, w_ref[...], preferred_element_type=jnp.float32) + bl_ref[...]
  col = lax.broadcasted_iota(jnp.int32, z.shape, 1)
  z = jnp.where(col < C, z, -1e30)
  m = jnp.max(z, axis=-1, keepdims=True)
  lse = jnp.log(jnp.sum(jnp.exp(z - m), axis=-1, keepdims=True))
  o_ref[...] = z - m - lse


def _tc_final(p0, p1, g, b, w, bl):
  return pl.pallas_call(
      _final_body,
      out_shape=jax.ShapeDtypeStruct((P, H), jnp.float32),
  )(p0, p1, g, b, w, bl)


def kernel(x, edge_index, output_node_mask, W0, ln0_g, ln0_b, W1, ln1_g,
           ln1_b, Wl, bl):
  xp = jnp.pad(x, ((0, NP - N), (0, 0)))
  dst = jnp.concatenate(
      [edge_index[0], jnp.full((E_PAD - E,), NP - 1, jnp.int32)])
  src = jnp.concatenate([edge_index[1], jnp.zeros((E_PAD - E,), jnp.int32)])
  packed = jnp.bitwise_or(src, jnp.left_shift(dst, 16))
  ed2d = packed.reshape(E_PAD // CHUNK, CHUNK)
  zeros = jnp.zeros((ROWS_PER_SUB, H), jnp.float32)
  dummy_mask = jnp.zeros((P,), jnp.int32)

  g0 = ln0_g.reshape(1, H)
  b0 = ln0_b.reshape(1, H)
  g1 = ln1_g.reshape(1, H)
  b1 = ln1_b.reshape(1, H)
  Wl_pad = jnp.pad(Wl, ((0, 0), (0, H - C)))
  bl_pad = jnp.pad(bl, ((0, H - C))).reshape(1, H)

  h = _tc_matmul(xp, W0)
  parts = _make_agg(False)(h, ed2d, zeros, dummy_mask)
  h2 = _tc_mid(parts[:NP], parts[NP:], g0, b0, W1)
  parts2 = _make_agg(True)(h2, ed2d, zeros, output_node_mask)
  outp = _tc_final(parts2[:P], parts2[P:], g1, b1, Wl_pad, bl_pad)
  return outp[:, :C]
